# Initial kernel scaffold; baseline (speedup 1.0000x reference)
import jax
import jax.numpy as jnp
from jax.experimental import pallas as pl


def kernel(x, edge_index, Wn1, Wr1, b1, Wn2, Wr2, b2):
    raise NotImplementedError("write your pallas kernel here")


# speedup vs baseline: 8.3783x; 8.3783x over previous
import functools

import jax
import jax.numpy as jnp
from jax import lax
from jax.experimental import pallas as pl
from jax.experimental.pallas import tpu as pltpu
from jax.experimental.pallas import tpu_sc as plsc

N = 10000
E = 320000
D = 128

NC = 2
NS = 16
NW = NC * NS
EPW = E // NW
CHUNK = 125
NCHUNK = EPW // CHUNK
NP = 10240
ROWS_PER_TILE = NP // NS


def _sc_agg_kernel(table, srcr, dstr, zrows, zcnt, ones_hbm,
                   pout, cout, accum, cnts, srcv, dstv, rows, onesv):
    c = lax.axis_index("c")
    s = lax.axis_index("s")
    w = c * NS + s

    pltpu.sync_copy(zrows, accum.at[pl.ds(s * ROWS_PER_TILE, ROWS_PER_TILE)])
    pltpu.sync_copy(zcnt, cnts.at[pl.ds(s * ROWS_PER_TILE, ROWS_PER_TILE)])
    pltpu.sync_copy(srcr.at[w], srcv)
    pltpu.sync_copy(dstr.at[w], dstv)
    pltpu.sync_copy(ones_hbm, onesv)
    plsc.subcore_barrier()

    def chunk(j, carry):
        pltpu.sync_copy(table.at[srcv.at[j]], rows)
        pltpu.sync_copy(rows, accum.at[dstv.at[j]], add=True)
        pltpu.sync_copy(onesv, cnts.at[dstv.at[j]], add=True)
        return carry

    lax.fori_loop(0, NCHUNK, chunk, 0)
    plsc.subcore_barrier()

    sl = pl.ds(s * ROWS_PER_TILE, ROWS_PER_TILE)
    pltpu.sync_copy(accum.at[sl], pout.at[c, sl])
    pltpu.sync_copy(cnts.at[sl], cout.at[c, sl])


@jax.jit
def _sc_agg(table, srcr, dstr):
    zrows = jnp.zeros((ROWS_PER_TILE, D), jnp.float32)
    zcnt = jnp.zeros((ROWS_PER_TILE,), jnp.float32)
    ones = jnp.ones((CHUNK,), jnp.float32)
    mesh = plsc.VectorSubcoreMesh(core_axis_name="c", subcore_axis_name="s")
    f = functools.partial(
        pl.kernel,
        out_type=[
            jax.ShapeDtypeStruct((NC, NP, D), jnp.float32),
            jax.ShapeDtypeStruct((NC, NP), jnp.float32),
        ],
        mesh=mesh,
        scratch_types=[
            pltpu.VMEM_SHARED((NP, D), jnp.float32),
            pltpu.VMEM_SHARED((NP,), jnp.float32),
            pltpu.VMEM((NCHUNK, CHUNK), jnp.int32),
            pltpu.VMEM((NCHUNK, CHUNK), jnp.int32),
            pltpu.VMEM((CHUNK, D), jnp.float32),
            pltpu.VMEM((CHUNK,), jnp.float32),
        ],
    )(_sc_agg_kernel)
    return f(table, srcr, dstr, zrows, zcnt, ones)


def _dense_kernel(p_ref, c_ref, x_ref, wn_ref, wr_ref, b_ref, o_ref, *, relu):
    agg = p_ref[0] + p_ref[1]
    cnt = c_ref[0, :, 0] + c_ref[1, :, 0]
    mean = agg * (1.0 / jnp.maximum(cnt, 1.0))[:, None]
    out = (jnp.dot(mean, wn_ref[...], precision=lax.Precision.HIGHEST)
           + jnp.dot(x_ref[...], wr_ref[...], precision=lax.Precision.HIGHEST)
           + b_ref[...][None, :])
    if relu:
        out = jnp.maximum(out, 0.0)
    o_ref[...] = out


def _dense(p, c, x, Wn, Wr, b, relu):
    BR = 1000
    grid = (N // BR,)
    return pl.pallas_call(
        functools.partial(_dense_kernel, relu=relu),
        grid=grid,
        in_specs=[
            pl.BlockSpec((NC, BR, D), lambda i: (0, i, 0)),
            pl.BlockSpec((NC, BR, 1), lambda i: (0, i, 0)),
            pl.BlockSpec((BR, D), lambda i: (i, 0)),
            pl.BlockSpec((D, D), lambda i: (0, 0)),
            pl.BlockSpec((D, D), lambda i: (0, 0)),
            pl.BlockSpec((D,), lambda i: (0,)),
        ],
        out_specs=pl.BlockSpec((BR, D), lambda i: (i, 0)),
        out_shape=jax.ShapeDtypeStruct((N, D), jnp.float32),
    )(p, c, x, Wn, Wr, b)


def kernel(x, edge_index, Wn1, Wr1, b1, Wn2, Wr2, b2):
    src = edge_index[0].astype(jnp.int32).reshape(NW, NCHUNK, CHUNK)
    dst = edge_index[1].astype(jnp.int32).reshape(NW, NCHUNK, CHUNK)

    p1, c1 = _sc_agg(x, src, dst)
    c1 = c1[..., None]
    h = _dense(p1, c1, x, Wn1, Wr1, b1, relu=True)
    p2, c2 = _sc_agg(h, src, dst)
    out = _dense(p2, c1, h, Wn2, Wr2, b2, relu=False)
    return out
